# Initial kernel scaffold; baseline (speedup 1.0000x reference)
import functools

import jax
import jax.numpy as jnp
from jax import lax
from jax.experimental import pallas as pl
from jax.experimental.pallas import tpu as pltpu, tpu_sc as plsc

D_HALF = 64
D = 128
N_TOTAL = 4096 * 50
NC, NS, L = 2, 16, 16
NW = NC * NS
N_PER_W = N_TOTAL // NW
CHUNK = 320
N_CHUNKS = N_PER_W // CHUNK
GROUPS = CHUNK // L


def _body(coords_hbm, rowtab_hbm, coltab_hbm, out_hbm,
          coords_v, rowtab_v, coltab_v, outbuf):
    wid = lax.axis_index("s") * NC + lax.axis_index("c")
    base_n = wid * N_PER_W

    pltpu.sync_copy(rowtab_hbm, rowtab_v)
    pltpu.sync_copy(coltab_hbm, coltab_v)
    pltpu.sync_copy(coords_hbm.at[pl.ds(base_n * 2, N_PER_W * 2)], coords_v)

    iota = lax.iota(jnp.int32, L)

    def chunk_body(ci, carry):
        def group_body(g, carry_g):
            n_loc = ci * CHUNK + g * L
            nvec = n_loc + iota
            r = plsc.load_gather(coords_v, [nvec * 2])
            c = plsc.load_gather(coords_v, [nvec * 2 + 1])
            r64 = jnp.clip(r, 0, D_HALF - 1) * D_HALF
            c64 = jnp.clip(c, 0, D_HALF - 1) * D_HALF
            obase = (g * L + iota) * D
            for d in range(D_HALF):
                rv = plsc.load_gather(rowtab_v, [r64 + d])
                plsc.store_scatter(outbuf, [obase + d], rv)
                cv = plsc.load_gather(coltab_v, [c64 + d])
                plsc.store_scatter(outbuf, [obase + D_HALF + d], cv)
            return carry_g

        lax.fori_loop(0, GROUPS, group_body, 0)
        pltpu.sync_copy(
            outbuf,
            out_hbm.at[pl.ds((base_n + ci * CHUNK) * D, CHUNK * D)])
        return carry

    lax.fori_loop(0, N_CHUNKS, chunk_body, 0)


_encode = functools.partial(
    pl.kernel,
    out_type=jax.ShapeDtypeStruct((N_TOTAL * D,), jnp.float32),
    mesh=plsc.VectorSubcoreMesh(core_axis_name="c", subcore_axis_name="s"),
    scratch_types=[
        pltpu.VMEM((N_PER_W * 2,), jnp.int32),
        pltpu.VMEM((D_HALF * D_HALF,), jnp.float32),
        pltpu.VMEM((D_HALF * D_HALF,), jnp.float32),
        pltpu.VMEM((CHUNK * D,), jnp.float32),
    ],
)(_body)


def kernel(coords, row_emb, col_emb):
    b, s, _ = coords.shape
    cf = coords.reshape(-1).astype(jnp.int32)
    out = _encode(cf, row_emb.reshape(-1), col_emb.reshape(-1))
    return out.reshape(b, s, D)


# speedup vs baseline: 1.7024x; 1.7024x over previous
import functools

import jax
import jax.numpy as jnp
from jax import lax
from jax.experimental import pallas as pl
from jax.experimental.pallas import tpu as pltpu, tpu_sc as plsc

D_HALF = 64
D = 128
N_TOTAL = 4096 * 50
NC, NS, L = 2, 16, 16
NW = NC * NS
N_PER_W = N_TOTAL // NW
CHUNK = 320
N_CHUNKS = N_PER_W // CHUNK
GROUPS = CHUNK // L


def _body(coords_hbm, rowtab_hbm, coltab_hbm, out_hbm,
          coords_v, rowtab_v, coltab_v, outbuf):
    wid = lax.axis_index("s") * NC + lax.axis_index("c")
    base_n = wid * N_PER_W

    pltpu.sync_copy(rowtab_hbm, rowtab_v)
    pltpu.sync_copy(coltab_hbm, coltab_v)
    pltpu.sync_copy(coords_hbm.at[pl.ds(base_n * 2, N_PER_W * 2)], coords_v)

    iota = lax.iota(jnp.int32, L)

    def chunk_body(ci, carry):
        def group_body(g, carry_g):
            n_loc = ci * CHUNK + g * L
            nvec = n_loc + iota
            r = plsc.load_gather(coords_v, [nvec * 2])
            c = plsc.load_gather(coords_v, [nvec * 2 + 1])
            r64 = jnp.clip(r, 0, D_HALF - 1) * D_HALF
            c64 = jnp.clip(c, 0, D_HALF - 1) * D_HALF
            obase = (g * L + iota) * D
            for d in range(D_HALF):
                rv = plsc.load_gather(rowtab_v, [r64 + d])
                plsc.store_scatter(outbuf, [obase + d], rv)
                cv = plsc.load_gather(coltab_v, [c64 + d])
                plsc.store_scatter(outbuf, [obase + D_HALF + d], cv)
            return carry_g

        lax.fori_loop(0, GROUPS, group_body, 0)
        pltpu.sync_copy(
            outbuf,
            out_hbm.at[pl.ds((base_n + ci * CHUNK) * D, CHUNK * D)])
        return carry

    lax.fori_loop(0, N_CHUNKS, chunk_body, 0)


_encode = functools.partial(
    pl.kernel,
    out_type=jax.ShapeDtypeStruct((N_TOTAL * D,), jnp.float32),
    mesh=plsc.VectorSubcoreMesh(core_axis_name="c", subcore_axis_name="s"),
    scratch_types=[
        pltpu.VMEM((N_PER_W * 2,), jnp.int32),
        pltpu.VMEM((D_HALF * D_HALF,), jnp.float32),
        pltpu.VMEM((D_HALF * D_HALF,), jnp.float32),
        pltpu.VMEM((CHUNK * D,), jnp.float32),
    ],
    compiler_params=pltpu.CompilerParams(needs_layout_passes=False),
)(_body)


def kernel(coords, row_emb, col_emb):
    b, s, _ = coords.shape
    cf = coords.reshape(-1).astype(jnp.int32)
    out = _encode(cf, row_emb.reshape(-1), col_emb.reshape(-1))
    return out.reshape(b, s, D)


# speedup vs baseline: 5.0174x; 2.9473x over previous
import functools

import jax
import jax.numpy as jnp
from jax import lax
from jax.experimental import pallas as pl
from jax.experimental.pallas import tpu as pltpu, tpu_sc as plsc

D_HALF = 64
D = 128
NV = D_HALF * D_HALF
N_TOTAL = 4096 * 50
NC, NS, L = 2, 16, 16
NW = NC * NS
N_PER_W = N_TOTAL // NW
CHUNK = 80
N_CHUNKS = N_PER_W // CHUNK
NBUF = 8
N_WAVES = N_CHUNKS // NBUF
GROUPS = N_PER_W // L
GPC = CHUNK // L


def _comb_body(rt_ref, ct_ref, out_ref):
    i0 = lax.broadcasted_iota(jnp.int32, (NV, D_HALF), 0)
    i1 = lax.broadcasted_iota(jnp.int32, (NV, D_HALF), 1)
    oh_r = (i0 // D_HALF == i1).astype(jnp.float32)
    oh_c = (i0 % D_HALF == i1).astype(jnp.float32)
    out_ref[:, :D_HALF] = jnp.dot(oh_r, rt_ref[...],
                                  preferred_element_type=jnp.float32)
    out_ref[:, D_HALF:] = jnp.dot(oh_c, ct_ref[...],
                                  preferred_element_type=jnp.float32)


_build_comb = pl.pallas_call(
    _comb_body,
    out_shape=jax.ShapeDtypeStruct((NV, D), jnp.float32),
)


def _body(coords_hbm, comb_hbm, out_hbm, coords_v, idx_v, *rest):
    bufs = rest[:NBUF]
    gsem = rest[NBUF:2 * NBUF]
    osem = rest[2 * NBUF:3 * NBUF]

    wid = lax.axis_index("s") * NC + lax.axis_index("c")
    base_n = wid * N_PER_W

    pltpu.sync_copy(coords_hbm.at[pl.ds(base_n * 2, N_PER_W * 2)], coords_v)

    iota = lax.iota(jnp.int32, L)

    def prep_body(g, carry):
        nvec = g * L + iota
        r = plsc.load_gather(coords_v, [nvec * 2])
        c = plsc.load_gather(coords_v, [nvec * 2 + 1])
        v = (jnp.clip(r, 0, D_HALF - 1) * D_HALF
             + jnp.clip(c, 0, D_HALF - 1))
        jrow = jnp.broadcast_to(g // GPC, (L,))
        kcol = (g % GPC) * L + iota
        plsc.store_scatter(idx_v, [jrow, kcol], v)
        return carry

    lax.fori_loop(0, GROUPS, prep_body, 0)

    def wave_body(i, carry):
        handles = []
        for b in range(NBUF):
            ci = i * NBUF + b

            @pl.when(i > 0)
            def _wait_out(b=b):
                pltpu.make_async_copy(
                    bufs[b], out_hbm.at[pl.ds(base_n, CHUNK)],
                    osem[b]).wait()

            handles.append(
                pltpu.async_copy(comb_hbm.at[idx_v.at[ci]], bufs[b], gsem[b]))
        for b in range(NBUF):
            ci = i * NBUF + b
            handles[b].wait()
            pltpu.async_copy(
                bufs[b], out_hbm.at[pl.ds(base_n + ci * CHUNK, CHUNK)],
                osem[b])
        return carry

    lax.fori_loop(0, N_WAVES, wave_body, 0)

    for b in range(NBUF):
        pltpu.make_async_copy(
            bufs[b], out_hbm.at[pl.ds(base_n, CHUNK)], osem[b]).wait()


_encode = functools.partial(
    pl.kernel,
    out_type=jax.ShapeDtypeStruct((N_TOTAL, D), jnp.float32),
    mesh=plsc.VectorSubcoreMesh(core_axis_name="c", subcore_axis_name="s"),
    scratch_types=(
        [pltpu.VMEM((N_PER_W * 2,), jnp.int32),
         pltpu.VMEM((N_CHUNKS, CHUNK), jnp.int32)]
        + [pltpu.VMEM((CHUNK, D), jnp.float32) for _ in range(NBUF)]
        + [pltpu.SemaphoreType.DMA for _ in range(2 * NBUF)]
    ),
    compiler_params=pltpu.CompilerParams(needs_layout_passes=False),
)(_body)


def kernel(coords, row_emb, col_emb):
    b, s, _ = coords.shape
    cf = coords.reshape(-1).astype(jnp.int32)
    comb = _build_comb(row_emb, col_emb)
    out = _encode(cf, comb)
    return out.reshape(b, s, D)


# speedup vs baseline: 7.0039x; 1.3959x over previous
import functools

import jax
import jax.numpy as jnp
from jax import lax
from jax.experimental import pallas as pl
from jax.experimental.pallas import tpu as pltpu, tpu_sc as plsc

D_HALF = 64
D = 128
NV = D_HALF * D_HALF
B, S = 4096, 50
N_TOTAL = B * S
NC, NS, L = 2, 16, 16
NW = NC * NS
N_PER_W = N_TOTAL // NW
CHUNK = S
N_CHUNKS = N_PER_W // CHUNK
NBUF = 8
N_WAVES = N_CHUNKS // NBUF
GROUPS = N_PER_W // L


def _comb_body(rt_ref, ct_ref, out_ref):
    i0 = lax.broadcasted_iota(jnp.int32, (NV, D_HALF), 0)
    i1 = lax.broadcasted_iota(jnp.int32, (NV, D_HALF), 1)
    oh_r = (i0 // D_HALF == i1).astype(jnp.float32)
    oh_c = (i0 % D_HALF == i1).astype(jnp.float32)
    out_ref[:, :D_HALF] = jnp.dot(oh_r, rt_ref[...],
                                  preferred_element_type=jnp.float32)
    out_ref[:, D_HALF:] = jnp.dot(oh_c, ct_ref[...],
                                  preferred_element_type=jnp.float32)


_build_comb = pl.pallas_call(
    _comb_body,
    out_shape=jax.ShapeDtypeStruct((NV, D), jnp.float32),
)


def _body(coords_hbm, comb_hbm, out_hbm, coords_v, idx_v, *rest):
    bufs = rest[:NBUF]
    gsem = rest[NBUF:2 * NBUF]
    osem = rest[2 * NBUF:3 * NBUF]

    wid = lax.axis_index("s") * NC + lax.axis_index("c")
    base_n = wid * N_PER_W
    base_b = wid * N_CHUNKS

    pltpu.sync_copy(coords_hbm.at[pl.ds(base_n * 2, N_PER_W * 2)], coords_v)

    iota = lax.iota(jnp.int32, L)

    def prep_body(g, carry):
        nvec = g * L + iota
        r = plsc.load_gather(coords_v, [nvec * 2])
        c = plsc.load_gather(coords_v, [nvec * 2 + 1])
        v = (jnp.clip(r, 0, D_HALF - 1) * D_HALF
             + jnp.clip(c, 0, D_HALF - 1))
        plsc.store_scatter(idx_v, [nvec // CHUNK, nvec % CHUNK], v)
        return carry

    lax.fori_loop(0, GROUPS, prep_body, 0)

    def wave_body(i, carry):
        handles = []
        for b in range(NBUF):
            ci = i * NBUF + b

            @pl.when(i > 0)
            def _wait_out(b=b):
                pltpu.make_async_copy(
                    bufs[b], out_hbm.at[base_b], osem[b]).wait()

            handles.append(
                pltpu.async_copy(comb_hbm.at[idx_v.at[ci]], bufs[b], gsem[b]))
        for b in range(NBUF):
            ci = i * NBUF + b
            handles[b].wait()
            pltpu.async_copy(bufs[b], out_hbm.at[base_b + ci], osem[b])
        return carry

    lax.fori_loop(0, N_WAVES, wave_body, 0)

    for b in range(NBUF):
        pltpu.make_async_copy(bufs[b], out_hbm.at[base_b], osem[b]).wait()


_encode = functools.partial(
    pl.kernel,
    out_type=jax.ShapeDtypeStruct((B, S, D), jnp.float32),
    mesh=plsc.VectorSubcoreMesh(core_axis_name="c", subcore_axis_name="s"),
    scratch_types=(
        [pltpu.VMEM((N_PER_W * 2,), jnp.int32),
         pltpu.VMEM((N_CHUNKS, CHUNK), jnp.int32)]
        + [pltpu.VMEM((CHUNK, D), jnp.float32) for _ in range(NBUF)]
        + [pltpu.SemaphoreType.DMA for _ in range(2 * NBUF)]
    ),
    compiler_params=pltpu.CompilerParams(needs_layout_passes=False),
)(_body)


def kernel(coords, row_emb, col_emb):
    cf = coords.reshape(-1).astype(jnp.int32)
    comb = _build_comb(row_emb, col_emb)
    return _encode(cf, comb)


# speedup vs baseline: 7.5230x; 1.0741x over previous
import functools

import jax
import jax.numpy as jnp
from jax import lax
from jax.experimental import pallas as pl
from jax.experimental.pallas import tpu as pltpu, tpu_sc as plsc

D_HALF = 64
D = 128
NV = D_HALF * D_HALF
B, S = 4096, 50
N_TOTAL = B * S
NC, NS, L = 2, 16, 16
NW = NC * NS
B_PER_W = B // NW
NBUF = 8
N_WAVES = B_PER_W // NBUF
IDX_BLK = 128


def _idx_body(c_ref, out_ref):
    c = c_ref[...].astype(jnp.int32)
    c = jnp.clip(c, 0, D_HALF - 1)
    k = lax.broadcasted_iota(jnp.int32, (IDX_BLK, S, 2), 2)
    w = jnp.where(k == 0, D_HALF, 1)
    out_ref[...] = jnp.sum(c * w, axis=-1)


def _make_idx(coords):
    return pl.pallas_call(
        _idx_body,
        grid=(B // IDX_BLK,),
        in_specs=[pl.BlockSpec((IDX_BLK, S, 2), lambda i: (i, 0, 0))],
        out_specs=pl.BlockSpec((IDX_BLK, S), lambda i: (i, 0)),
        out_shape=jax.ShapeDtypeStruct((B, S), jnp.int32),
    )(coords)


def _comb_body(rt_ref, ct_ref, out_ref):
    i0 = lax.broadcasted_iota(jnp.int32, (NV, D_HALF), 0)
    i1 = lax.broadcasted_iota(jnp.int32, (NV, D_HALF), 1)
    oh_r = (i0 // D_HALF == i1).astype(jnp.float32)
    oh_c = (i0 % D_HALF == i1).astype(jnp.float32)
    out_ref[:, :D_HALF] = jnp.dot(oh_r, rt_ref[...],
                                  preferred_element_type=jnp.float32)
    out_ref[:, D_HALF:] = jnp.dot(oh_c, ct_ref[...],
                                  preferred_element_type=jnp.float32)


_build_comb = pl.pallas_call(
    _comb_body,
    out_shape=jax.ShapeDtypeStruct((NV, D), jnp.float32),
)


def _body(idx_hbm, comb_hbm, out_hbm, idx_v, *rest):
    bufs = rest[:NBUF]
    gsem = rest[NBUF:2 * NBUF]
    osem = rest[2 * NBUF:3 * NBUF]

    wid = lax.axis_index("s") * NC + lax.axis_index("c")
    base_b = wid * B_PER_W

    pltpu.sync_copy(idx_hbm.at[pl.ds(base_b, B_PER_W)], idx_v)

    def wave_body(i, carry):
        handles = []
        for b in range(NBUF):
            ci = i * NBUF + b

            @pl.when(i > 0)
            def _wait_out(b=b):
                pltpu.make_async_copy(
                    bufs[b], out_hbm.at[base_b], osem[b]).wait()

            handles.append(
                pltpu.async_copy(comb_hbm.at[idx_v.at[ci]], bufs[b], gsem[b]))
        for b in range(NBUF):
            ci = i * NBUF + b
            handles[b].wait()
            pltpu.async_copy(bufs[b], out_hbm.at[base_b + ci], osem[b])
        return carry

    lax.fori_loop(0, N_WAVES, wave_body, 0)

    for b in range(NBUF):
        pltpu.make_async_copy(bufs[b], out_hbm.at[base_b], osem[b]).wait()


_encode = functools.partial(
    pl.kernel,
    out_type=jax.ShapeDtypeStruct((B, S, D), jnp.float32),
    mesh=plsc.VectorSubcoreMesh(core_axis_name="c", subcore_axis_name="s"),
    scratch_types=(
        [pltpu.VMEM((B_PER_W, S), jnp.int32)]
        + [pltpu.VMEM((S, D), jnp.float32) for _ in range(NBUF)]
        + [pltpu.SemaphoreType.DMA for _ in range(2 * NBUF)]
    ),
    compiler_params=pltpu.CompilerParams(
        needs_layout_passes=False, use_tc_tiling_on_sc=True),
)(_body)


def kernel(coords, row_emb, col_emb):
    idx = _make_idx(coords)
    comb = _build_comb(row_emb, col_emb)
    return _encode(idx, comb)
